# baseline (speedup 1.0000x reference)
import functools

import jax
import jax.numpy as jnp
import numpy as np
from jax import lax
from jax.experimental import pallas as pl
from jax.experimental.pallas import tpu as pltpu
from jax.experimental.pallas import tpu_sc as plsc

_M = 7
_SHIFT = 23 - _M
_K = 0x3F800000 >> _SHIFT
_N = 224 * 224
_NROWS = 16 * 96
_NC, _NS, _L = 2, 16, 16
_NW = _NC * _NS
_RPW = _NROWS // _NW
_WIN = 25088
_NWIN = _N // _WIN
_CPW = _WIN // _L
_NCH = _N // _L
_KCH = _K // _L
_GRP = _NCH // _L
_STEP_LEN = _N + _L

_EXP_TABLE = np.zeros((256,), np.float32)
_EXP_TABLE[:255] = np.ldexp(np.float32(1.0), np.arange(255) - 127)
_C0 = np.float32(1.0 + 0.5 / (1 << _M))
_C1 = np.float32(1.0 / (1 << _M))

_mesh = plsc.VectorSubcoreMesh(
    core_axis_name="c", subcore_axis_name="s",
    num_cores=_NC, num_subcores=_NS)


def _body(x_hbm, scale_hbm, etbl_hbm, out_hbm,
          hist, step, aux1, aux2, iwin, owin, etbl, sbuf):
  wid = lax.axis_index("s") * _NC + lax.axis_index("c")
  lanes = lax.iota(jnp.int32, _L)
  zeros_i = jnp.zeros((_L,), jnp.int32)
  zeros_f = jnp.zeros((_L,), jnp.float32)
  ones_i = jnp.ones((_L,), jnp.int32)

  pltpu.sync_copy(scale_hbm, sbuf)
  scale_v = sbuf[...]
  pltpu.sync_copy(etbl_hbm, etbl)

  def _zero_hist(i, carry):
    hist[pl.ds(i * _L, _L)] = zeros_i
    return carry

  def _zero_step(i, carry):
    step[pl.ds(i * _L, _L)] = zeros_f
    return carry

  lax.fori_loop(0, _KCH, _zero_hist, 0)
  lax.fori_loop(0, _STEP_LEN // _L, _zero_step, 0)

  def row_loop(i, carry):
    row = wid * _RPW + i

    def win_loop(w, c):
      pltpu.sync_copy(x_hbm.at[pl.ds(row * _N + w * _WIN, _WIN)], iwin)

      def h_loop(j, cc):
        keys = lax.shift_right_logical(iwin[pl.ds(j * _L, _L)], _SHIFT)
        plsc.addupdate_scatter(hist, [keys], ones_i)
        return cc

      return c

    lax.fori_loop(0, _NWIN, win_loop, 0)

    def b_loop(j, cbase):
      c = hist[pl.ds(j * _L, _L)]
      hist[pl.ds(j * _L, _L)] = zeros_i
      incl = plsc.cumsum(c)
      pos = cbase + incl - c
      kv = j * _L + lanes
      e_idx = lax.shift_right_logical(kv, _M)
      mant = jnp.bitwise_and(kv, (1 << _M) - 1)
      se = plsc.load_gather(etbl, [e_idx])
      v = se * (mant.astype(jnp.float32) * _C1 + _C0)
      plsc.addupdate_scatter(step, [pos], v)
      plsc.addupdate_scatter(step, [pos + c], -v)
      return cbase + jnp.max(incl)

    lax.fori_loop(0, _KCH, b_loop, jnp.int32(0))

    lane0 = lanes == 0

    def a_loop(j, c):
      mch = step[pl.ds(j * _L, _L)]
      loc = plsc.cumsum(mch)
      step[pl.ds(j * _L, _L)] = loc
      idx = jnp.full((_L,), j, jnp.int32)
      s1 = jnp.full((_L,), jnp.max(loc))
      s2 = jnp.full((_L,), jnp.sum(loc))
      plsc.store_scatter(aux1, [idx], s1, mask=lane0)
      plsc.store_scatter(aux2, [idx], s2, mask=lane0)
      return c

    lax.fori_loop(0, _NCH, a_loop, 0, unroll=4)

    def g_loop(g, carry):
      cb1, cb2 = carry
      a = aux1[pl.ds(g * _L, _L)]
      incl1 = plsc.cumsum(a) + cb1
      excl1 = incl1 - a
      aux1[pl.ds(g * _L, _L)] = excl1
      a2 = aux2[pl.ds(g * _L, _L)]
      cs2 = a2 + jnp.float32(_L) * excl1
      incl2 = plsc.cumsum(cs2) + cb2
      aux2[pl.ds(g * _L, _L)] = incl2 - cs2
      return jnp.max(incl1), jnp.max(incl2)

    _, total = lax.fori_loop(
        0, _GRP, g_loop, (jnp.float32(0.0), jnp.float32(0.0)))
    inv_t = scale_v / jnp.full((_L,), total)

    def ow_loop(w, c):
      def e_loop(jj, cc):
        j = w * _CPW + jj
        idx = jnp.full((_L,), j, jnp.int32)
        c1 = plsc.load_gather(aux1, [idx])
        c2 = plsc.load_gather(aux2, [idx])
        loc = step[pl.ds(j * _L, _L)]
        sorted_q = loc + c1
        o = (plsc.cumsum(sorted_q) + c2) * inv_t
        step[pl.ds(j * _L, _L)] = zeros_f
        owin[pl.ds(jj * _L, _L)] = o
        return cc

      lax.fori_loop(0, _CPW, e_loop, c, unroll=4)
      pltpu.sync_copy(owin, out_hbm.at[pl.ds(row * _N + w * _WIN, _WIN)])
      return c

    lax.fori_loop(0, _NWIN, ow_loop, 0)
    return carry

  lax.fori_loop(0, _RPW, row_loop, 0)


_cdf_sc = functools.partial(
    pl.kernel,
    out_type=jax.ShapeDtypeStruct((_NROWS * _N,), jnp.float32),
    mesh=_mesh,
    compiler_params=pltpu.CompilerParams(needs_layout_passes=False),
    scratch_types=[
        pltpu.VMEM((_K,), jnp.int32),
        pltpu.VMEM((_STEP_LEN,), jnp.float32),
        pltpu.VMEM((_NCH,), jnp.float32),
        pltpu.VMEM((_NCH,), jnp.float32),
        pltpu.VMEM((_WIN,), jnp.int32),
        pltpu.VMEM((_WIN,), jnp.float32),
        pltpu.VMEM((256,), jnp.float32),
        pltpu.VMEM((_L,), jnp.float32),
    ],
)(_body)


def kernel(x, scale):
  b, c, h, w = x.shape
  xi = lax.bitcast_convert_type(x.reshape(b * c * h * w), jnp.int32)
  scale_v = jnp.full((_L,), scale, jnp.float32)
  etbl = jnp.asarray(_EXP_TABLE)
  out = _cdf_sc(xi, scale_v, etbl)
  return out.reshape(b, c, h, w)


# speedup vs baseline: 1.1579x; 1.1579x over previous
import functools

import jax
import jax.numpy as jnp
import numpy as np
from jax import lax
from jax.experimental import pallas as pl
from jax.experimental.pallas import tpu as pltpu
from jax.experimental.pallas import tpu_sc as plsc

_M = 7
_SHIFT = 23 - _M
_K = 0x3F800000 >> _SHIFT
_N = 224 * 224
_NROWS = 16 * 96
_NC, _NS, _L = 2, 16, 16
_NW = _NC * _NS
_RPW = _NROWS // _NW
_WIN = 25088
_NWIN = _N // _WIN
_CPW = _WIN // _L
_NCH = _N // _L
_KCH = _K // _L
_GRP = _NCH // _L
_STEP_LEN = _N + _L

_EXP_TABLE = np.zeros((256,), np.float32)
_EXP_TABLE[:255] = np.ldexp(np.float32(1.0), np.arange(255) - 127)
_C0 = np.float32(1.0 + 0.5 / (1 << _M))
_C1 = np.float32(1.0 / (1 << _M))

_mesh = plsc.VectorSubcoreMesh(
    core_axis_name="c", subcore_axis_name="s",
    num_cores=_NC, num_subcores=_NS)


def _body(x_hbm, scale_hbm, etbl_hbm, out_hbm,
          hist, step, aux1, aux2, iwin, owin, etbl, sbuf):
  wid = lax.axis_index("s") * _NC + lax.axis_index("c")
  lanes = lax.iota(jnp.int32, _L)
  zeros_i = jnp.zeros((_L,), jnp.int32)
  zeros_f = jnp.zeros((_L,), jnp.float32)
  ones_i = jnp.ones((_L,), jnp.int32)

  pltpu.sync_copy(scale_hbm, sbuf)
  scale_v = sbuf[...]
  pltpu.sync_copy(etbl_hbm, etbl)

  def _zero_hist(i, carry):
    hist[pl.ds(i * _L, _L)] = zeros_i
    return carry

  def _zero_step(i, carry):
    step[pl.ds(i * _L, _L)] = zeros_f
    return carry

  lax.fori_loop(0, _KCH, _zero_hist, 0)
  lax.fori_loop(0, _STEP_LEN // _L, _zero_step, 0)

  def row_loop(i, carry):
    row = wid * _RPW + i

    def win_loop(w, c):
      pltpu.sync_copy(x_hbm.at[pl.ds(row * _N + w * _WIN, _WIN)], iwin)

      def h_loop(j, cc):
        keys = lax.shift_right_logical(iwin[pl.ds(j * _L, _L)], _SHIFT)
        plsc.addupdate_scatter(hist, [keys], ones_i)
        return cc

      lax.fori_loop(0, _CPW, h_loop, c, unroll=4)
      return c

    lax.fori_loop(0, _NWIN, win_loop, 0)

    def b_loop(j, cbase):
      c = hist[pl.ds(j * _L, _L)]
      hist[pl.ds(j * _L, _L)] = zeros_i
      incl = plsc.cumsum(c)
      pos = cbase + incl - c
      kv = j * _L + lanes
      e_idx = lax.shift_right_logical(kv, _M)
      mant = jnp.bitwise_and(kv, (1 << _M) - 1)
      se = plsc.load_gather(etbl, [e_idx])
      v = se * (mant.astype(jnp.float32) * _C1 + _C0)
      plsc.addupdate_scatter(step, [pos], v)
      plsc.addupdate_scatter(step, [pos + c], -v)
      return cbase + jnp.sum(c)

    lax.fori_loop(0, _KCH, b_loop, jnp.int32(0))

    def ag_loop(g, c):
      acc1 = zeros_f
      acc2 = zeros_f
      for k in range(_L):
        j = g * _L + k
        mch = step[pl.ds(j * _L, _L)]
        loc = plsc.cumsum(mch)
        step[pl.ds(j * _L, _L)] = loc
        lk = lanes == k
        acc1 = jnp.where(lk, jnp.max(loc), acc1)
        acc2 = jnp.where(lk, jnp.sum(loc), acc2)
      aux1[pl.ds(g * _L, _L)] = acc1
      aux2[pl.ds(g * _L, _L)] = acc2
      return c

    lax.fori_loop(0, _GRP, ag_loop, 0)

    def g_loop(g, carry):
      cb1, cb2 = carry
      a = aux1[pl.ds(g * _L, _L)]
      incl1 = plsc.cumsum(a) + cb1
      excl1 = incl1 - a
      aux1[pl.ds(g * _L, _L)] = excl1
      a2 = aux2[pl.ds(g * _L, _L)]
      cs2 = a2 + jnp.float32(_L) * excl1
      incl2 = plsc.cumsum(cs2) + cb2
      aux2[pl.ds(g * _L, _L)] = incl2 - cs2
      return jnp.max(incl1), jnp.max(incl2)

    _, total = lax.fori_loop(
        0, _GRP, g_loop, (jnp.float32(0.0), jnp.float32(0.0)))
    inv_t = scale_v / jnp.full((_L,), total)

    def ow_loop(w, c):
      def eg_loop(gg, cc):
        g = w * (_CPW // _L) + gg
        a1v = aux1[pl.ds(g * _L, _L)]
        a2v = aux2[pl.ds(g * _L, _L)]
        for k in range(_L):
          j = g * _L + k
          jj = gg * _L + k
          loc = step[pl.ds(j * _L, _L)]
          sorted_q = loc + jnp.full((_L,), a1v[k])
          o = (plsc.cumsum(sorted_q) + jnp.full((_L,), a2v[k])) * inv_t
          step[pl.ds(j * _L, _L)] = zeros_f
          owin[pl.ds(jj * _L, _L)] = o
        return cc

      lax.fori_loop(0, _CPW // _L, eg_loop, c)
      pltpu.sync_copy(owin, out_hbm.at[pl.ds(row * _N + w * _WIN, _WIN)])
      return c

    lax.fori_loop(0, _NWIN, ow_loop, 0)
    return carry

  lax.fori_loop(0, _RPW, row_loop, 0)


_cdf_sc = functools.partial(
    pl.kernel,
    out_type=jax.ShapeDtypeStruct((_NROWS * _N,), jnp.float32),
    mesh=_mesh,
    compiler_params=pltpu.CompilerParams(needs_layout_passes=False),
    scratch_types=[
        pltpu.VMEM((_K,), jnp.int32),
        pltpu.VMEM((_STEP_LEN,), jnp.float32),
        pltpu.VMEM((_NCH,), jnp.float32),
        pltpu.VMEM((_NCH,), jnp.float32),
        pltpu.VMEM((_WIN,), jnp.int32),
        pltpu.VMEM((_WIN,), jnp.float32),
        pltpu.VMEM((256,), jnp.float32),
        pltpu.VMEM((_L,), jnp.float32),
    ],
)(_body)


def kernel(x, scale):
  b, c, h, w = x.shape
  xi = lax.bitcast_convert_type(x.reshape(b * c * h * w), jnp.int32)
  scale_v = jnp.full((_L,), scale, jnp.float32)
  etbl = jnp.asarray(_EXP_TABLE)
  out = _cdf_sc(xi, scale_v, etbl)
  return out.reshape(b, c, h, w)


# speedup vs baseline: 1.3128x; 1.1338x over previous
import functools

import jax
import jax.numpy as jnp
import numpy as np
from jax import lax
from jax.experimental import pallas as pl
from jax.experimental.pallas import tpu as pltpu
from jax.experimental.pallas import tpu_sc as plsc

_M = 7
_SHIFT = 23 - _M
_K = 0x3F800000 >> _SHIFT
_N = 224 * 224
_NROWS = 16 * 96
_NC, _NS, _L = 2, 16, 16
_NW = _NC * _NS
_RPW = _NROWS // _NW
_WIN = 25088
_NWIN = _N // _WIN
_CPW = _WIN // _L
_NCH = _N // _L
_KCH = _K // _L
_GRP = _NCH // _L
_STEP_LEN = _N + _L

_EXP_TABLE = np.zeros((256,), np.float32)
_EXP_TABLE[:255] = np.ldexp(np.float32(1.0), np.arange(255) - 127)
_C0 = np.float32(1.0 + 0.5 / (1 << _M))
_C1 = np.float32(1.0 / (1 << _M))

_mesh = plsc.VectorSubcoreMesh(
    core_axis_name="c", subcore_axis_name="s",
    num_cores=_NC, num_subcores=_NS)


def _body(x_hbm, scale_hbm, etbl_hbm, out_hbm,
          hist, step, aux1, aux2, iwin, owin, etbl, sbuf):
  wid = lax.axis_index("s") * _NC + lax.axis_index("c")
  lanes = lax.iota(jnp.int32, _L)
  zeros_i = jnp.zeros((_L,), jnp.int32)
  zeros_f = jnp.zeros((_L,), jnp.float32)
  ones_i = jnp.ones((_L,), jnp.int32)

  pltpu.sync_copy(scale_hbm, sbuf)
  scale_v = sbuf[...]
  pltpu.sync_copy(etbl_hbm, etbl)

  def _zero_hist(i, carry):
    hist[pl.ds(i * _L, _L)] = zeros_i
    return carry

  def _zero_step(i, carry):
    step[pl.ds(i * _L, _L)] = zeros_f
    return carry

  lax.fori_loop(0, _KCH, _zero_hist, 0)
  lax.fori_loop(0, _STEP_LEN // _L, _zero_step, 0)

  def row_loop(i, carry):
    row = wid * _RPW + i

    def win_loop(w, c):
      pltpu.sync_copy(x_hbm.at[pl.ds(row * _N + w * _WIN, _WIN)], iwin)

      def h_loop(j, cc):
        keys = lax.shift_right_logical(iwin[pl.ds(j * _L, _L)], _SHIFT)
        plsc.addupdate_scatter(hist, [keys], ones_i)
        return cc

      lax.fori_loop(0, _CPW, h_loop, c, unroll=4)
      return c

    lax.fori_loop(0, _NWIN, win_loop, 0)

    def b_loop(j, cbase):
      c = hist[pl.ds(j * _L, _L)]
      hist[pl.ds(j * _L, _L)] = zeros_i
      tot = jnp.sum(c)

      @pl.when(tot > 0)
      def _():
        incl = plsc.cumsum(c)
        pos = cbase + incl - c
        kv = j * _L + lanes
        e_idx = lax.shift_right_logical(kv, _M)
        mant = jnp.bitwise_and(kv, (1 << _M) - 1)
        se = plsc.load_gather(etbl, [e_idx])
        v = se * (mant.astype(jnp.float32) * _C1 + _C0)
        plsc.addupdate_scatter(step, [pos], v)
        plsc.addupdate_scatter(step, [pos + c], -v)

      return cbase + tot

    lax.fori_loop(0, _KCH, b_loop, jnp.int32(0))

    def ag_loop(g, c):
      acc1 = zeros_f
      acc2 = zeros_f
      for k in range(_L):
        j = g * _L + k
        mch = step[pl.ds(j * _L, _L)]
        loc = plsc.cumsum(mch)
        step[pl.ds(j * _L, _L)] = loc
        lk = lanes == k
        acc1 = jnp.where(lk, loc[_L - 1], acc1)
        acc2 = jnp.where(lk, jnp.sum(loc), acc2)
      aux1[pl.ds(g * _L, _L)] = acc1
      aux2[pl.ds(g * _L, _L)] = acc2
      return c

    lax.fori_loop(0, _GRP, ag_loop, 0)

    def g_loop(g, carry):
      cb1, cb2 = carry
      a = aux1[pl.ds(g * _L, _L)]
      incl1 = plsc.cumsum(a) + cb1
      excl1 = incl1 - a
      aux1[pl.ds(g * _L, _L)] = excl1
      a2 = aux2[pl.ds(g * _L, _L)]
      cs2 = a2 + jnp.float32(_L) * excl1
      incl2 = plsc.cumsum(cs2) + cb2
      aux2[pl.ds(g * _L, _L)] = incl2 - cs2
      return jnp.max(incl1), jnp.max(incl2)

    _, total = lax.fori_loop(
        0, _GRP, g_loop, (jnp.float32(0.0), jnp.float32(0.0)))
    inv_t = scale_v / jnp.full((_L,), total)

    def ow_loop(w, c):
      def eg_loop(gg, cc):
        g = w * (_CPW // _L) + gg
        a1v = aux1[pl.ds(g * _L, _L)]
        a2v = aux2[pl.ds(g * _L, _L)]
        for k in range(_L):
          j = g * _L + k
          jj = gg * _L + k
          loc = step[pl.ds(j * _L, _L)]
          sorted_q = loc + jnp.full((_L,), a1v[k])
          o = (plsc.cumsum(sorted_q) + jnp.full((_L,), a2v[k])) * inv_t
          step[pl.ds(j * _L, _L)] = zeros_f
          owin[pl.ds(jj * _L, _L)] = o
        return cc

      lax.fori_loop(0, _CPW // _L, eg_loop, c)
      pltpu.sync_copy(owin, out_hbm.at[pl.ds(row * _N + w * _WIN, _WIN)])
      return c

    lax.fori_loop(0, _NWIN, ow_loop, 0)
    return carry

  lax.fori_loop(0, _RPW, row_loop, 0)


_cdf_sc = functools.partial(
    pl.kernel,
    out_type=jax.ShapeDtypeStruct((_NROWS * _N,), jnp.float32),
    mesh=_mesh,
    compiler_params=pltpu.CompilerParams(needs_layout_passes=False),
    scratch_types=[
        pltpu.VMEM((_K,), jnp.int32),
        pltpu.VMEM((_STEP_LEN,), jnp.float32),
        pltpu.VMEM((_NCH,), jnp.float32),
        pltpu.VMEM((_NCH,), jnp.float32),
        pltpu.VMEM((_WIN,), jnp.int32),
        pltpu.VMEM((_WIN,), jnp.float32),
        pltpu.VMEM((256,), jnp.float32),
        pltpu.VMEM((_L,), jnp.float32),
    ],
)(_body)


def kernel(x, scale):
  b, c, h, w = x.shape
  xi = lax.bitcast_convert_type(x.reshape(b * c * h * w), jnp.int32)
  scale_v = jnp.full((_L,), scale, jnp.float32)
  etbl = jnp.asarray(_EXP_TABLE)
  out = _cdf_sc(xi, scale_v, etbl)
  return out.reshape(b, c, h, w)
